# baseline (speedup 1.0000x reference)
import functools

import jax
import jax.numpy as jnp
from jax import lax
from jax.experimental import pallas as pl
from jax.experimental.pallas import tpu as pltpu
from jax.experimental.pallas import tpu_sc as plsc

N_GOS = 100000
D = 128
B = 16384
MARGIN_ = 0.1
EPS = 1e-5

NC = 2
NS = 16
NW = NC * NS
PAIRS_W = B // NW
CH = 128
NCH = PAIRS_W // CH

def _sc_gather_body(emb, rad, idx0, idx1, c_out, d_out, rc_out, rd_out,
                    idx_v, rows_v, rad_v, sem):
    wid = lax.axis_index("s") * NC + lax.axis_index("c")
    base = wid * PAIRS_W
    pltpu.sync_copy(idx0.at[pl.ds(wid * NCH, NCH)], idx_v.at[pl.ds(0, NCH)])
    pltpu.sync_copy(idx1.at[pl.ds(wid * NCH, NCH)], idx_v.at[pl.ds(NCH, NCH)])
    for j in range(NCH):
        pltpu.async_copy(emb.at[idx_v.at[j]], rows_v, sem).wait()
        pltpu.sync_copy(rows_v, c_out.at[pl.ds(base + j * CH, CH)])
        pltpu.async_copy(rad.at[idx_v.at[j]], rad_v, sem).wait()
        pltpu.sync_copy(rad_v, rc_out.at[pl.ds(base + j * CH, CH)])
    for j in range(NCH):
        pltpu.async_copy(emb.at[idx_v.at[NCH + j]], rows_v, sem).wait()
        pltpu.sync_copy(rows_v, d_out.at[pl.ds(base + j * CH, CH)])
        pltpu.async_copy(rad.at[idx_v.at[NCH + j]], rad_v, sem).wait()
        pltpu.sync_copy(rad_v, rd_out.at[pl.ds(base + j * CH, CH)])


@functools.lru_cache(maxsize=1)
def _make_sc_gather():
    mesh = plsc.VectorSubcoreMesh(core_axis_name="c", subcore_axis_name="s")
    return pl.kernel(
        _sc_gather_body,
        out_type=(
            jax.ShapeDtypeStruct((B, D), jnp.float32),
            jax.ShapeDtypeStruct((B, D), jnp.float32),
            jax.ShapeDtypeStruct((B, 1), jnp.float32),
            jax.ShapeDtypeStruct((B, 1), jnp.float32),
        ),
        mesh=mesh,
        scratch_types=[
            pltpu.VMEM((2 * NCH, CH), jnp.int32),
            pltpu.VMEM((CH, D), jnp.float32),
            pltpu.VMEM((CH, 1), jnp.float32),
            pltpu.SemaphoreType.DMA,
        ],
        compiler_params=pltpu.CompilerParams(use_tc_tiling_on_sc=False),
    )


NB = 16
BR = B // NB


def _tc_body(c_ref, d_ref, rc_ref, rd_ref, g_ref, out_ref, stats_v, acc_v):
    p = pl.program_id(0)
    j = pl.program_id(1)

    @pl.when((p == 0) & (j == 0))
    def _init():
        stats_v[...] = jnp.zeros_like(stats_v)
        acc_v[0, 0] = jnp.float32(0.0)

    @pl.when(p == 0)
    def _stats():
        c = c_ref[...]
        d = d_ref[...]
        stats_v[0:1, :] += jnp.sum(c, axis=0, keepdims=True)
        stats_v[1:2, :] += jnp.sum(c * c, axis=0, keepdims=True)
        stats_v[2:3, :] += jnp.sum(d, axis=0, keepdims=True)
        stats_v[3:4, :] += jnp.sum(d * d, axis=0, keepdims=True)

    @pl.when((p == 1) & (j == 0))
    def _finalize_stats():
        n = jnp.float32(B)
        g = g_ref[...]
        mu_c = stats_v[0:1, :] / n
        var_c = stats_v[1:2, :] / n - mu_c * mu_c
        mu_d = stats_v[2:3, :] / n
        var_d = stats_v[3:4, :] / n - mu_d * mu_d
        inv_c = g / jnp.sqrt(var_c + EPS)
        inv_d = g / jnp.sqrt(var_d + EPS)
        stats_v[4:5, :] = inv_c
        stats_v[5:6, :] = inv_d
        stats_v[6:7, :] = mu_d * inv_d - mu_c * inv_c

    @pl.when(p == 1)
    def _loss():
        a = stats_v[4:5, :]
        bb = stats_v[5:6, :]
        off = stats_v[6:7, :]
        x = c_ref[...] * a - d_ref[...] * bb + off
        s = jnp.sum(x * x, axis=1, keepdims=True)
        dist = jnp.sqrt(s) + jnp.abs(rc_ref[...]) - jnp.abs(rd_ref[...])
        h = jnp.maximum(dist - MARGIN_, 0.0)
        acc_v[0, 0] += jnp.sum(h)

    @pl.when((p == 1) & (j == NB - 1))
    def _emit():
        out_ref[...] = jnp.full((1, 1), acc_v[0, 0] / jnp.float32(B), jnp.float32)


_tc_finalize = pl.pallas_call(
    _tc_body,
    grid=(2, NB),
    in_specs=[
        pl.BlockSpec((BR, D), lambda p, j: (j, 0)),
        pl.BlockSpec((BR, D), lambda p, j: (j, 0)),
        pl.BlockSpec((BR, 1), lambda p, j: (j, 0)),
        pl.BlockSpec((BR, 1), lambda p, j: (j, 0)),
        pl.BlockSpec((1, D), lambda p, j: (0, 0)),
    ],
    out_specs=pl.BlockSpec((1, 1), lambda p, j: (0, 0)),
    out_shape=jax.ShapeDtypeStruct((1, 1), jnp.float32),
    scratch_shapes=[
        pltpu.VMEM((8, D), jnp.float32),
        pltpu.SMEM((1, 1), jnp.float32),
    ],
)


def kernel(data, go_embed_weight, go_rad_weight, bn_weight, bn_bias):
    del bn_bias
    idx0 = data[:, 0].reshape(B // CH, CH)
    idx1 = data[:, 1].reshape(B // CH, CH)
    c_raw, d_raw, rc, rd = _make_sc_gather()(go_embed_weight, go_rad_weight, idx0, idx1)
    loss = _tc_finalize(c_raw, d_raw, rc, rd, bn_weight.reshape(1, D))
    return loss[0, 0]


# speedup vs baseline: 1.1848x; 1.1848x over previous
import functools

import jax
import jax.numpy as jnp
from jax import lax
from jax.experimental import pallas as pl
from jax.experimental.pallas import tpu as pltpu
from jax.experimental.pallas import tpu_sc as plsc

N_GOS = 100000
D = 128
B = 16384
MARGIN_ = 0.1
EPS = 1e-5

NC = 2
NS = 16
NW = NC * NS
PAIRS_W = B // NW
CH = 128
NCH = PAIRS_W // CH
NCHT = 2 * NCH
LANES = 16


def _sc_gather_body(emb, rad, idx0, idx1, c_out, d_out, rc_out, rd_out,
                    stats_out, idx_v, rows_v, rad_v, stats_v, sems):
    wid = lax.axis_index("s") * NC + lax.axis_index("c")
    base = wid * PAIRS_W
    pltpu.sync_copy(idx0.at[pl.ds(wid * NCH, NCH)], idx_v.at[pl.ds(0, NCH)])
    pltpu.sync_copy(idx1.at[pl.ds(wid * NCH, NCH)], idx_v.at[pl.ds(NCH, NCH)])

    rad_dmas = []
    for j in range(NCHT):
        rad_dmas.append(
            pltpu.async_copy(rad.at[idx_v.at[j]],
                             rad_v.at[pl.ds(j * CH, CH)], sems.at[4]))

    zeros = [jnp.zeros((LANES,), jnp.float32) for _ in range(2 * (D // LANES))]

    def chunk_stats(rv, acc):
        def row(r, acc):
            out = []
            for k in range(D // LANES):
                v = rv[r, pl.ds(k * LANES, LANES)]
                out.append(acc[k] + v)
            for k in range(D // LANES):
                v = rv[r, pl.ds(k * LANES, LANES)]
                out.append(acc[D // LANES + k] + v * v)
            return tuple(out)
        return lax.fori_loop(0, CH, row, tuple(acc), unroll=2)

    outs = (c_out, d_out)
    gat = [None, None]
    wrb = [None, None]
    acc_c = list(zeros)
    acc_d = list(zeros)
    gat[0] = pltpu.async_copy(emb.at[idx_v.at[0]], rows_v.at[0], sems.at[0])
    for j in range(NCHT):
        b = j % 2
        nb = (j + 1) % 2
        if j + 1 < NCHT:
            if wrb[nb] is not None:
                wrb[nb].wait()
            gat[nb] = pltpu.async_copy(emb.at[idx_v.at[j + 1]],
                                       rows_v.at[nb], sems.at[nb])
        gat[b].wait()
        if j < NCH:
            acc_c = list(chunk_stats(rows_v.at[b], acc_c))
        else:
            acc_d = list(chunk_stats(rows_v.at[b], acc_d))
        dst = outs[j // NCH].at[pl.ds(base + (j % NCH) * CH, CH)]
        wrb[b] = pltpu.async_copy(rows_v.at[b], dst, sems.at[2 + b])

    for k in range(D // LANES):
        stats_v[0, pl.ds(k * LANES, LANES)] = acc_c[k]
        stats_v[1, pl.ds(k * LANES, LANES)] = acc_c[D // LANES + k]
        stats_v[2, pl.ds(k * LANES, LANES)] = acc_d[k]
        stats_v[3, pl.ds(k * LANES, LANES)] = acc_d[D // LANES + k]
    for k in range(4):
        pltpu.sync_copy(stats_v.at[k], stats_out.at[k * NW + wid])

    for dma in rad_dmas:
        dma.wait()
    pltpu.sync_copy(rad_v.at[pl.ds(0, PAIRS_W)], rc_out.at[pl.ds(base, PAIRS_W)])
    pltpu.sync_copy(rad_v.at[pl.ds(PAIRS_W, PAIRS_W)], rd_out.at[pl.ds(base, PAIRS_W)])
    wrb[0].wait()
    wrb[1].wait()


@functools.lru_cache(maxsize=1)
def _make_sc_gather():
    mesh = plsc.VectorSubcoreMesh(core_axis_name="c", subcore_axis_name="s")
    return pl.kernel(
        _sc_gather_body,
        out_type=(
            jax.ShapeDtypeStruct((B, D), jnp.float32),
            jax.ShapeDtypeStruct((B, D), jnp.float32),
            jax.ShapeDtypeStruct((B, 1), jnp.float32),
            jax.ShapeDtypeStruct((B, 1), jnp.float32),
            jax.ShapeDtypeStruct((4 * NW, D), jnp.float32),
        ),
        mesh=mesh,
        scratch_types=[
            pltpu.VMEM((NCHT, CH), jnp.int32),
            pltpu.VMEM((2, CH, D), jnp.float32),
            pltpu.VMEM((NCHT * CH, 1), jnp.float32),
            pltpu.VMEM((4, D), jnp.float32),
            pltpu.SemaphoreType.DMA((8,)),
        ],
        compiler_params=pltpu.CompilerParams(use_tc_tiling_on_sc=False),
    )


NB = 8
BR = B // NB


def _tc_body(c_ref, d_ref, rc_ref, rd_ref, stats_ref, g_ref, out_ref,
             coef_v, acc_v):
    j = pl.program_id(0)

    @pl.when(j == 0)
    def _prep():
        n = jnp.float32(B)
        g = g_ref[...]
        sum_c = jnp.sum(stats_ref[0 * NW:1 * NW, :], axis=0, keepdims=True)
        sq_c = jnp.sum(stats_ref[1 * NW:2 * NW, :], axis=0, keepdims=True)
        sum_d = jnp.sum(stats_ref[2 * NW:3 * NW, :], axis=0, keepdims=True)
        sq_d = jnp.sum(stats_ref[3 * NW:4 * NW, :], axis=0, keepdims=True)
        mu_c = sum_c / n
        var_c = sq_c / n - mu_c * mu_c
        mu_d = sum_d / n
        var_d = sq_d / n - mu_d * mu_d
        inv_c = g / jnp.sqrt(var_c + EPS)
        inv_d = g / jnp.sqrt(var_d + EPS)
        coef_v[0:1, :] = inv_c
        coef_v[1:2, :] = inv_d
        coef_v[2:3, :] = mu_d * inv_d - mu_c * inv_c
        acc_v[0, 0] = jnp.float32(0.0)

    a = coef_v[0:1, :]
    bb = coef_v[1:2, :]
    off = coef_v[2:3, :]
    x = c_ref[...] * a - d_ref[...] * bb + off
    s = jnp.sum(x * x, axis=1, keepdims=True)
    dist = jnp.sqrt(s) + jnp.abs(rc_ref[...]) - jnp.abs(rd_ref[...])
    h = jnp.maximum(dist - MARGIN_, 0.0)
    acc_v[0, 0] += jnp.sum(h)

    @pl.when(j == NB - 1)
    def _emit():
        out_ref[...] = jnp.full((1, 1), acc_v[0, 0] / jnp.float32(B), jnp.float32)


_tc_finalize = pl.pallas_call(
    _tc_body,
    grid=(NB,),
    in_specs=[
        pl.BlockSpec((BR, D), lambda j: (j, 0)),
        pl.BlockSpec((BR, D), lambda j: (j, 0)),
        pl.BlockSpec((BR, 1), lambda j: (j, 0)),
        pl.BlockSpec((BR, 1), lambda j: (j, 0)),
        pl.BlockSpec((4 * NW, D), lambda j: (0, 0)),
        pl.BlockSpec((1, D), lambda j: (0, 0)),
    ],
    out_specs=pl.BlockSpec((1, 1), lambda j: (0, 0)),
    out_shape=jax.ShapeDtypeStruct((1, 1), jnp.float32),
    scratch_shapes=[
        pltpu.VMEM((4, D), jnp.float32),
        pltpu.SMEM((1, 1), jnp.float32),
    ],
)


def kernel(data, go_embed_weight, go_rad_weight, bn_weight, bn_bias):
    del bn_bias
    idx0 = data[:, 0].reshape(B // CH, CH)
    idx1 = data[:, 1].reshape(B // CH, CH)
    c_raw, d_raw, rc, rd, stats = _make_sc_gather()(
        go_embed_weight, go_rad_weight, idx0, idx1)
    loss = _tc_finalize(c_raw, d_raw, rc, rd, stats, bn_weight.reshape(1, D))
    return loss[0, 0]


# speedup vs baseline: 3.5614x; 3.0059x over previous
import functools

import jax
import jax.numpy as jnp
from jax import lax
from jax.experimental import pallas as pl
from jax.experimental.pallas import tpu as pltpu
from jax.experimental.pallas import tpu_sc as plsc

N_GOS = 100000
D = 128
B = 16384
MARGIN_ = 0.1
EPS = 1e-5

NC = 2
NS = 16
NW = NC * NS
PAIRS_W = B // NW
CH = 128
NCH = PAIRS_W // CH
NCHT = 2 * NCH
LANES = 16


def _sc_gather_body(emb, rad, idx_all, c_out, d_out, rc_out, rd_out,
                    stats_out, idx_v, rows_v, rad_v, stats_v, sems):
    wid = lax.axis_index("s") * NC + lax.axis_index("c")
    base = wid * PAIRS_W
    pltpu.sync_copy(idx_all.at[wid], idx_v)

    rad_dmas = []
    for j in range(NCHT):
        rad_dmas.append(
            pltpu.async_copy(rad.at[idx_v.at[j]], rad_v.at[j], sems.at[4]))

    zeros = [jnp.zeros((LANES,), jnp.float32) for _ in range(2 * (D // LANES))]

    def chunk_stats(rv, acc):
        def row(r, acc):
            out = []
            for k in range(D // LANES):
                v = rv[r, pl.ds(k * LANES, LANES)]
                out.append(acc[k] + v)
            for k in range(D // LANES):
                v = rv[r, pl.ds(k * LANES, LANES)]
                out.append(acc[D // LANES + k] + v * v)
            return tuple(out)
        return lax.fori_loop(0, CH, row, tuple(acc), unroll=2)

    outs = (c_out, d_out)
    gat = [None, None]
    wrb = [None, None]
    acc_c = list(zeros)
    acc_d = list(zeros)
    gat[0] = pltpu.async_copy(emb.at[idx_v.at[0]], rows_v.at[0], sems.at[0])
    for j in range(NCHT):
        b = j % 2
        nb = (j + 1) % 2
        if j + 1 < NCHT:
            if wrb[nb] is not None:
                wrb[nb].wait()
            gat[nb] = pltpu.async_copy(emb.at[idx_v.at[j + 1]],
                                       rows_v.at[nb], sems.at[nb])
        gat[b].wait()
        if j < NCH:
            acc_c = list(chunk_stats(rows_v.at[b], acc_c))
        else:
            acc_d = list(chunk_stats(rows_v.at[b], acc_d))
        dst = outs[j // NCH].at[pl.ds(base + (j % NCH) * CH, CH)]
        wrb[b] = pltpu.async_copy(rows_v.at[b], dst, sems.at[2 + b])

    for k in range(D // LANES):
        stats_v[0, pl.ds(k * LANES, LANES)] = acc_c[k]
        stats_v[1, pl.ds(k * LANES, LANES)] = acc_c[D // LANES + k]
        stats_v[2, pl.ds(k * LANES, LANES)] = acc_d[k]
        stats_v[3, pl.ds(k * LANES, LANES)] = acc_d[D // LANES + k]
    pltpu.sync_copy(stats_v, stats_out.at[wid])

    for dma in rad_dmas:
        dma.wait()
    pltpu.sync_copy(rad_v.at[pl.ds(0, NCH)], rc_out.at[wid])
    pltpu.sync_copy(rad_v.at[pl.ds(NCH, NCH)], rd_out.at[wid])
    wrb[0].wait()
    wrb[1].wait()


@functools.lru_cache(maxsize=1)
def _make_sc_gather():
    mesh = plsc.VectorSubcoreMesh(core_axis_name="c", subcore_axis_name="s")
    return pl.kernel(
        _sc_gather_body,
        out_type=(
            jax.ShapeDtypeStruct((B, D), jnp.float32),
            jax.ShapeDtypeStruct((B, D), jnp.float32),
            jax.ShapeDtypeStruct((NW, NCH, CH), jnp.float32),
            jax.ShapeDtypeStruct((NW, NCH, CH), jnp.float32),
            jax.ShapeDtypeStruct((NW, 4, D), jnp.float32),
        ),
        mesh=mesh,
        scratch_types=[
            pltpu.VMEM((NCHT, CH), jnp.int32),
            pltpu.VMEM((2, CH, D), jnp.float32),
            pltpu.VMEM((NCHT, CH), jnp.float32),
            pltpu.VMEM((4, D), jnp.float32),
            pltpu.SemaphoreType.DMA((8,)),
        ],
    )


NB = 8
BR = B // NB
WPB = NW // NB


def _tc_body(c_ref, d_ref, rc_ref, rd_ref, stats_ref, g_ref, out_ref,
             coef_v, acc_v):
    j = pl.program_id(0)

    @pl.when(j == 0)
    def _prep():
        n = jnp.float32(B)
        g = g_ref[...]
        sum_c = jnp.sum(stats_ref[:, 0:1, :], axis=0)
        sq_c = jnp.sum(stats_ref[:, 1:2, :], axis=0)
        sum_d = jnp.sum(stats_ref[:, 2:3, :], axis=0)
        sq_d = jnp.sum(stats_ref[:, 3:4, :], axis=0)
        mu_c = sum_c / n
        var_c = sq_c / n - mu_c * mu_c
        mu_d = sum_d / n
        var_d = sq_d / n - mu_d * mu_d
        inv_c = g / jnp.sqrt(var_c + EPS)
        inv_d = g / jnp.sqrt(var_d + EPS)
        coef_v[0:1, :] = inv_c
        coef_v[1:2, :] = inv_d
        coef_v[2:3, :] = mu_d * inv_d - mu_c * inv_c
        acc_v[0, 0] = jnp.float32(0.0)

    a = coef_v[0:1, :]
    bb = coef_v[1:2, :]
    off = coef_v[2:3, :]
    x = c_ref[...] * a - d_ref[...] * bb + off
    s = jnp.sum(x * x, axis=1)
    sm = s.reshape(BR // D, D)
    rca = jnp.abs(rc_ref[...].reshape(BR // D, D))
    rda = jnp.abs(rd_ref[...].reshape(BR // D, D))
    dist = jnp.sqrt(sm) + rca - rda - MARGIN_
    acc_v[0, 0] += jnp.sum(jnp.maximum(dist, 0.0))

    @pl.when(j == NB - 1)
    def _emit():
        out_ref[...] = jnp.full((1, 1), acc_v[0, 0] / jnp.float32(B), jnp.float32)


_tc_finalize = pl.pallas_call(
    _tc_body,
    grid=(NB,),
    in_specs=[
        pl.BlockSpec((BR, D), lambda j: (j, 0)),
        pl.BlockSpec((BR, D), lambda j: (j, 0)),
        pl.BlockSpec((WPB, NCH, CH), lambda j: (j, 0, 0)),
        pl.BlockSpec((WPB, NCH, CH), lambda j: (j, 0, 0)),
        pl.BlockSpec((NW, 4, D), lambda j: (0, 0, 0)),
        pl.BlockSpec((1, D), lambda j: (0, 0)),
    ],
    out_specs=pl.BlockSpec((1, 1), lambda j: (0, 0)),
    out_shape=jax.ShapeDtypeStruct((1, 1), jnp.float32),
    scratch_shapes=[
        pltpu.VMEM((4, D), jnp.float32),
        pltpu.SMEM((1, 1), jnp.float32),
    ],
)


def kernel(data, go_embed_weight, go_rad_weight, bn_weight, bn_bias):
    del bn_bias
    idx0 = data[:, 0].reshape(NW, NCH, CH)
    idx1 = data[:, 1].reshape(NW, NCH, CH)
    idx_all = jnp.concatenate([idx0, idx1], axis=1)
    rad1 = go_rad_weight.reshape(N_GOS)
    c_raw, d_raw, rc, rd, stats = _make_sc_gather()(
        go_embed_weight, rad1, idx_all)
    loss = _tc_finalize(c_raw, d_raw, rc, rd, stats, bn_weight.reshape(1, D))
    return loss[0, 0]


# speedup vs baseline: 3.7686x; 1.0582x over previous
import functools

import jax
import jax.numpy as jnp
from jax import lax
from jax.experimental import pallas as pl
from jax.experimental.pallas import tpu as pltpu
from jax.experimental.pallas import tpu_sc as plsc

N_GOS = 100000
D = 128
B = 16384
MARGIN_ = 0.1
EPS = 1e-5

NC = 2
NS = 16
NW = NC * NS
PAIRS_W = B // NW
CH = 128
NCH = PAIRS_W // CH
NCHT = 2 * NCH
LANES = 16


def _sc_gather_body(emb, rad, idx_all, c_out, d_out, radd_out,
                    stats_out, idx_v, rows_v, rad_v, radd_v, stats_v, sems):
    wid = lax.axis_index("s") * NC + lax.axis_index("c")
    base = wid * PAIRS_W
    pltpu.sync_copy(idx_all.at[wid], idx_v)

    rad_dmas = []
    for j in range(NCHT):
        rad_dmas.append(
            pltpu.async_copy(rad.at[idx_v.at[j]], rad_v.at[j], sems.at[4]))

    zeros = [jnp.zeros((LANES,), jnp.float32) for _ in range(2 * (D // LANES))]

    def chunk_stats(rv, acc):
        def row(r, acc):
            out = []
            for k in range(D // LANES):
                v = rv[r, pl.ds(k * LANES, LANES)]
                out.append(acc[k] + v)
            for k in range(D // LANES):
                v = rv[r, pl.ds(k * LANES, LANES)]
                out.append(acc[D // LANES + k] + v * v)
            return tuple(out)
        return lax.fori_loop(0, CH, row, tuple(acc), unroll=2)

    outs = (c_out, d_out)
    gat = [None, None]
    wrb = [None, None]
    acc_c = list(zeros)
    acc_d = list(zeros)
    gat[0] = pltpu.async_copy(emb.at[idx_v.at[0]], rows_v.at[0], sems.at[0])
    for j in range(NCHT):
        b = j % 2
        nb = (j + 1) % 2
        if j + 1 < NCHT:
            if wrb[nb] is not None:
                wrb[nb].wait()
            gat[nb] = pltpu.async_copy(emb.at[idx_v.at[j + 1]],
                                       rows_v.at[nb], sems.at[nb])
        gat[b].wait()
        if j < NCH:
            acc_c = list(chunk_stats(rows_v.at[b], acc_c))
        else:
            acc_d = list(chunk_stats(rows_v.at[b], acc_d))
        dst = outs[j // NCH].at[pl.ds(base + (j % NCH) * CH, CH)]
        wrb[b] = pltpu.async_copy(rows_v.at[b], dst, sems.at[2 + b])

    for k in range(D // LANES):
        stats_v[0, pl.ds(k * LANES, LANES)] = acc_c[k]
        stats_v[1, pl.ds(k * LANES, LANES)] = acc_c[D // LANES + k]
        stats_v[2, pl.ds(k * LANES, LANES)] = acc_d[k]
        stats_v[3, pl.ds(k * LANES, LANES)] = acc_d[D // LANES + k]
    pltpu.sync_copy(stats_v, stats_out.at[wid])

    for dma in rad_dmas:
        dma.wait()
    for j in range(NCH):
        for k in range(CH // LANES):
            sl = pl.ds(k * LANES, LANES)
            radd_v[j, sl] = jnp.abs(rad_v[j, sl]) - jnp.abs(rad_v[NCH + j, sl])
    pltpu.sync_copy(radd_v, radd_out.at[wid])
    wrb[0].wait()
    wrb[1].wait()


@functools.lru_cache(maxsize=1)
def _make_sc_gather():
    mesh = plsc.VectorSubcoreMesh(core_axis_name="c", subcore_axis_name="s")
    return pl.kernel(
        _sc_gather_body,
        out_type=(
            jax.ShapeDtypeStruct((B, D), jnp.float32),
            jax.ShapeDtypeStruct((B, D), jnp.float32),
            jax.ShapeDtypeStruct((NW, NCH, CH), jnp.float32),
            jax.ShapeDtypeStruct((NW, 4, D), jnp.float32),
        ),
        mesh=mesh,
        scratch_types=[
            pltpu.VMEM((NCHT, CH), jnp.int32),
            pltpu.VMEM((2, CH, D), jnp.float32),
            pltpu.VMEM((NCHT, CH), jnp.float32),
            pltpu.VMEM((NCH, CH), jnp.float32),
            pltpu.VMEM((4, D), jnp.float32),
            pltpu.SemaphoreType.DMA((8,)),
        ],
    )


NB = 4
BR = B // NB
WPB = NW // NB


def _tc_body(c_ref, d_ref, radd_ref, stats_ref, g_ref, out_ref,
             coef_v, acc_v):
    j = pl.program_id(0)

    @pl.when(j == 0)
    def _prep():
        n = jnp.float32(B)
        g = g_ref[...]
        sum_c = jnp.sum(stats_ref[:, 0:1, :], axis=0)
        sq_c = jnp.sum(stats_ref[:, 1:2, :], axis=0)
        sum_d = jnp.sum(stats_ref[:, 2:3, :], axis=0)
        sq_d = jnp.sum(stats_ref[:, 3:4, :], axis=0)
        mu_c = sum_c / n
        var_c = sq_c / n - mu_c * mu_c
        mu_d = sum_d / n
        var_d = sq_d / n - mu_d * mu_d
        inv_c = g / jnp.sqrt(var_c + EPS)
        inv_d = g / jnp.sqrt(var_d + EPS)
        coef_v[0:1, :] = inv_c
        coef_v[1:2, :] = inv_d
        coef_v[2:3, :] = mu_d * inv_d - mu_c * inv_c
        acc_v[0, 0] = jnp.float32(0.0)

    a = coef_v[0:1, :]
    bb = coef_v[1:2, :]
    off = coef_v[2:3, :]
    x = c_ref[...] * a - d_ref[...] * bb + off
    s = jnp.sum(x * x, axis=1)
    sm = s.reshape(BR // D, D)
    ra = radd_ref[...].reshape(BR // D, D)
    dist = jnp.sqrt(sm) + ra - MARGIN_
    acc_v[0, 0] += jnp.sum(jnp.maximum(dist, 0.0))

    @pl.when(j == NB - 1)
    def _emit():
        out_ref[...] = jnp.full((1, 1), acc_v[0, 0] / jnp.float32(B), jnp.float32)


_tc_finalize = pl.pallas_call(
    _tc_body,
    grid=(NB,),
    in_specs=[
        pl.BlockSpec((BR, D), lambda j: (j, 0)),
        pl.BlockSpec((BR, D), lambda j: (j, 0)),
        pl.BlockSpec((WPB, NCH, CH), lambda j: (j, 0, 0)),
        pl.BlockSpec((NW, 4, D), lambda j: (0, 0, 0)),
        pl.BlockSpec((1, D), lambda j: (0, 0)),
    ],
    out_specs=pl.BlockSpec((1, 1), lambda j: (0, 0)),
    out_shape=jax.ShapeDtypeStruct((1, 1), jnp.float32),
    scratch_shapes=[
        pltpu.VMEM((4, D), jnp.float32),
        pltpu.SMEM((1, 1), jnp.float32),
    ],
)


def kernel(data, go_embed_weight, go_rad_weight, bn_weight, bn_bias):
    del bn_bias
    idx_all = (data.reshape(NW, NCH, CH, 2)
               .transpose(0, 3, 1, 2)
               .reshape(NW, NCHT, CH))
    rad1 = go_rad_weight.reshape(N_GOS)
    c_raw, d_raw, radd, stats = _make_sc_gather()(
        go_embed_weight, rad1, idx_all)
    loss = _tc_finalize(c_raw, d_raw, radd, stats, bn_weight.reshape(1, D))
    return loss[0, 0]
